# baseline (speedup 1.0000x reference)
import functools

import jax
import jax.numpy as jnp
from jax import lax
from jax.experimental import pallas as pl
from jax.experimental.pallas import tpu as pltpu
from jax.experimental.pallas import tpu_sc as plsc

N = 10000
D = 128
E = 320000
NC, NS = 2, 16
NW = NC * NS
CH = 128
NCHUNK = E // CH
PW = 80
NPAD = NW * PW
NBUF = 2
SB = 40
NSB = PW // SB
N2 = 10240
RPT = N2 // NS

_mesh = plsc.VectorSubcoreMesh(core_axis_name="c", subcore_axis_name="s")


@functools.partial(
    pl.kernel,
    out_type=jax.ShapeDtypeStruct((NC, N2), jnp.float32),
    mesh=_mesh,
    scratch_types=[
        pltpu.VMEM((PW, CH), jnp.int32),
        pltpu.VMEM((CH,), jnp.float32),
        pltpu.VMEM_SHARED((N2,), jnp.float32),
        pltpu.SemaphoreType.DMA,
    ],
)
def _sc_deg(dst2_hbm, ones_hbm, zeros_hbm, out_hbm, dst_rows, ones_v, dacc,
            sem):
    c = lax.axis_index("c")
    s = lax.axis_index("s")
    w = s * NC + c
    rbase = s * RPT
    pltpu.sync_copy(zeros_hbm.at[pl.ds(rbase, RPT)], dacc.at[pl.ds(rbase, RPT)])
    pltpu.sync_copy(dst2_hbm.at[pl.ds(w * PW, PW)], dst_rows)
    pltpu.sync_copy(ones_hbm, ones_v)
    plsc.subcore_barrier()

    def fire(i, carry):
        pltpu.async_copy(ones_v, dacc.at[dst_rows.at[i]], sem, add=True)
        return carry

    lax.fori_loop(0, PW, fire, 0)

    def drain(i, carry):
        pltpu.make_async_copy(ones_v, dacc.at[dst_rows.at[0]], sem).wait()
        return carry

    lax.fori_loop(0, PW, drain, 0)
    plsc.subcore_barrier()
    pltpu.sync_copy(dacc.at[pl.ds(rbase, RPT)],
                    out_hbm.at[c].at[pl.ds(rbase, RPT)])


@functools.partial(
    pl.kernel,
    out_type=jax.ShapeDtypeStruct((NC, N2, D), jnp.float32),
    mesh=_mesh,
    scratch_types=[
        pltpu.VMEM((SB, CH), jnp.int32),
        pltpu.VMEM((SB, CH), jnp.int32),
        pltpu.VMEM((NBUF, CH, D), jnp.float32),
        pltpu.VMEM_SHARED((N2, D), jnp.float32),
        pltpu.SemaphoreType.DMA((NBUF,)),
        pltpu.SemaphoreType.DMA((NBUF,)),
    ],
)
def _sc_msg(t_hbm, src2_hbm, dst2_hbm, zeros_hbm, out_hbm,
            src_rows, dst_rows, rows, acc, gsem, ssem):
    c = lax.axis_index("c")
    s = lax.axis_index("s")
    w = s * NC + c
    rbase = s * RPT
    pltpu.sync_copy(zeros_hbm.at[pl.ds(rbase, RPT)], acc.at[pl.ds(rbase, RPT)])
    plsc.subcore_barrier()

    for sc in range(NSB):
        base = w * PW + sc * SB
        pltpu.sync_copy(src2_hbm.at[pl.ds(base, SB)], src_rows)
        pltpu.sync_copy(dst2_hbm.at[pl.ds(base, SB)], dst_rows)
        for b in range(NBUF):
            pltpu.async_copy(t_hbm.at[src_rows.at[b]], rows.at[b],
                             gsem.at[b])

        def body(g, carry):
            for b in range(NBUF):
                i = g * NBUF + b
                pltpu.make_async_copy(t_hbm.at[src_rows.at[i]], rows.at[b],
                                      gsem.at[b]).wait()
                pltpu.async_copy(rows.at[b], acc.at[dst_rows.at[i]],
                                 ssem.at[b], add=True)

                @pl.when(i + NBUF < SB)
                def _():
                    pltpu.make_async_copy(rows.at[b],
                                          acc.at[dst_rows.at[i]],
                                          ssem.at[b]).wait()
                    pltpu.async_copy(t_hbm.at[src_rows.at[i + NBUF]],
                                     rows.at[b], gsem.at[b])

            return carry

        lax.fori_loop(0, SB // NBUF, body, 0)
        for b in range(NBUF):
            pltpu.make_async_copy(rows.at[b], acc.at[dst_rows.at[0]],
                                  ssem.at[b]).wait()

    plsc.subcore_barrier()
    pltpu.sync_copy(acc.at[pl.ds(rbase, RPT)],
                    out_hbm.at[c].at[pl.ds(rbase, RPT)])


R = 1000
GRID = N // R

_PREC = lax.Precision.HIGHEST


def _tc_scale_body(degp_ref, x_ref, t1_ref, dinv_ref):
    deg = degp_ref[:, 0] + degp_ref[:, 1] + 1.0
    dinv = lax.rsqrt(deg)
    dinv_ref[...] = dinv[:, None]
    t1_ref[...] = dinv[:, None] * x_ref[...]


def _tc_scale(degp_t, x):
    return pl.pallas_call(
        _tc_scale_body,
        grid=(GRID,),
        in_specs=[
            pl.BlockSpec((R, 2), lambda i: (i, 0)),
            pl.BlockSpec((R, D), lambda i: (i, 0)),
        ],
        out_specs=[
            pl.BlockSpec((R, D), lambda i: (i, 0)),
            pl.BlockSpec((R, 1), lambda i: (i, 0)),
        ],
        out_shape=[
            jax.ShapeDtypeStruct((N, D), jnp.float32),
            jax.ShapeDtypeStruct((N, 1), jnp.float32),
        ],
    )(degp_t, x)


def _tc_mid_body(y1p_ref, t1_ref, dinv_ref, w1_ref, b1_ref, w2_ref, t2_ref):
    dinv = dinv_ref[...]
    z1 = dinv * (y1p_ref[0] + y1p_ref[1] + t1_ref[...])
    h = jnp.maximum(
        jnp.dot(z1, w1_ref[...], preferred_element_type=jnp.float32,
                precision=_PREC) + b1_ref[...],
        0.0)
    s2 = jnp.dot(h, w2_ref[...], preferred_element_type=jnp.float32,
                 precision=_PREC)
    t2_ref[...] = dinv * s2


def _tc_mid(y1p, t1, dinv, W1, b1, W2):
    return pl.pallas_call(
        _tc_mid_body,
        grid=(GRID,),
        in_specs=[
            pl.BlockSpec((2, R, D), lambda i: (0, i, 0)),
            pl.BlockSpec((R, D), lambda i: (i, 0)),
            pl.BlockSpec((R, 1), lambda i: (i, 0)),
            pl.BlockSpec((D, 2 * D), lambda i: (0, 0)),
            pl.BlockSpec((1, 2 * D), lambda i: (0, 0)),
            pl.BlockSpec((2 * D, D), lambda i: (0, 0)),
        ],
        out_specs=pl.BlockSpec((R, D), lambda i: (i, 0)),
        out_shape=jax.ShapeDtypeStruct((N, D), jnp.float32),
    )(y1p, t1, dinv, W1, b1, W2)


def _tc_out_body(y2p_ref, t2_ref, dinv_ref, b2_ref, out_ref):
    out_ref[...] = dinv_ref[...] * (y2p_ref[0] + y2p_ref[1] + t2_ref[...]) \
        + b2_ref[...]


def _tc_out(y2p, t2, dinv, b2):
    return pl.pallas_call(
        _tc_out_body,
        grid=(GRID,),
        in_specs=[
            pl.BlockSpec((2, R, D), lambda i: (0, i, 0)),
            pl.BlockSpec((R, D), lambda i: (i, 0)),
            pl.BlockSpec((R, 1), lambda i: (i, 0)),
            pl.BlockSpec((1, D), lambda i: (0, 0)),
        ],
        out_specs=pl.BlockSpec((R, D), lambda i: (i, 0)),
        out_shape=jax.ShapeDtypeStruct((N, D), jnp.float32),
    )(y2p, t2, dinv, b2)


def kernel(x, edge_index, W1, b1, W2, b2):
    src2 = edge_index[0].astype(jnp.int32).reshape(NCHUNK, CH)
    dst2 = edge_index[1].astype(jnp.int32).reshape(NCHUNK, CH)
    src2 = jnp.concatenate(
        [src2, jnp.zeros((NPAD - NCHUNK, CH), jnp.int32)])
    dst2 = jnp.concatenate(
        [dst2, jnp.full((NPAD - NCHUNK, CH), N, jnp.int32)])
    zeros_nd = jnp.zeros((N2, D), jnp.float32)
    zeros_n = jnp.zeros((N2,), jnp.float32)
    ones_ch = jnp.ones((CH,), jnp.float32)

    degp = _sc_deg(dst2, ones_ch, zeros_n)
    t1, dinv = _tc_scale(degp.T, x)
    y1p = _sc_msg(t1, src2, dst2, zeros_nd)
    t2 = _tc_mid(y1p, t1, dinv, W1, b1.reshape(1, -1), W2)
    y2p = _sc_msg(t2, src2, dst2, zeros_nd)
    return _tc_out(y2p, t2, dinv, b2.reshape(1, -1))


# speedup vs baseline: 3.4680x; 3.4680x over previous
import functools

import jax
import jax.numpy as jnp
from jax import lax
from jax.experimental import pallas as pl
from jax.experimental.pallas import tpu as pltpu
from jax.experimental.pallas import tpu_sc as plsc

N = 10000
D = 128
E = 320000
NC, NS = 2, 16
NW = NC * NS
CH = 64
NCHUNK = E // CH
PW = 160
NPAD = NW * PW
NBUF = 4
SB = 40
NSB = PW // SB
N2 = 10240
RPT = N2 // NS

_mesh = plsc.VectorSubcoreMesh(core_axis_name="c", subcore_axis_name="s")


@functools.partial(
    pl.kernel,
    out_type=jax.ShapeDtypeStruct((NC, N2), jnp.float32),
    mesh=_mesh,
    scratch_types=[
        pltpu.VMEM((PW, CH), jnp.int32),
        pltpu.VMEM((CH,), jnp.float32),
        pltpu.VMEM_SHARED((N2,), jnp.float32),
        pltpu.SemaphoreType.DMA,
    ],
)
def _sc_deg(dst2_hbm, ones_hbm, zeros_hbm, out_hbm, dst_rows, ones_v, dacc,
            sem):
    c = lax.axis_index("c")
    s = lax.axis_index("s")
    w = s * NC + c
    rbase = s * RPT
    pltpu.sync_copy(zeros_hbm.at[pl.ds(rbase, RPT)], dacc.at[pl.ds(rbase, RPT)])
    pltpu.sync_copy(dst2_hbm.at[pl.ds(w * PW, PW)], dst_rows)
    pltpu.sync_copy(ones_hbm, ones_v)
    plsc.subcore_barrier()

    def fire(i, carry):
        pltpu.async_copy(ones_v, dacc.at[dst_rows.at[i]], sem, add=True)
        return carry

    lax.fori_loop(0, PW, fire, 0)

    def drain(i, carry):
        pltpu.make_async_copy(ones_v, dacc.at[dst_rows.at[0]], sem).wait()
        return carry

    lax.fori_loop(0, PW, drain, 0)
    plsc.subcore_barrier()
    pltpu.sync_copy(dacc.at[pl.ds(rbase, RPT)],
                    out_hbm.at[c].at[pl.ds(rbase, RPT)])


@functools.partial(
    pl.kernel,
    out_type=jax.ShapeDtypeStruct((NC, N2, D), jnp.float32),
    mesh=_mesh,
    scratch_types=[
        pltpu.VMEM((SB, CH), jnp.int32),
        pltpu.VMEM((SB, CH), jnp.int32),
        pltpu.VMEM((NBUF, CH, D), jnp.float32),
        pltpu.VMEM_SHARED((N2, D), jnp.float32),
        pltpu.SemaphoreType.DMA((NBUF,)),
        pltpu.SemaphoreType.DMA((NBUF,)),
    ],
)
def _sc_msg(t_hbm, src2_hbm, dst2_hbm, zeros_hbm, out_hbm,
            src_rows, dst_rows, rows, acc, gsem, ssem):
    c = lax.axis_index("c")
    s = lax.axis_index("s")
    w = s * NC + c
    rbase = s * RPT
    pltpu.sync_copy(zeros_hbm.at[pl.ds(rbase, RPT)], acc.at[pl.ds(rbase, RPT)])
    plsc.subcore_barrier()

    for sc in range(NSB):
        base = w * PW + sc * SB
        pltpu.sync_copy(src2_hbm.at[pl.ds(base, SB)], src_rows)
        pltpu.sync_copy(dst2_hbm.at[pl.ds(base, SB)], dst_rows)
        for b in range(NBUF):
            pltpu.async_copy(t_hbm.at[src_rows.at[b]], rows.at[b],
                             gsem.at[b])

        def body(g, carry):
            for b in range(NBUF):
                i = g * NBUF + b
                pltpu.make_async_copy(t_hbm.at[src_rows.at[i]], rows.at[b],
                                      gsem.at[b]).wait()
                pltpu.async_copy(rows.at[b], acc.at[dst_rows.at[i]],
                                 ssem.at[b], add=True)

                @pl.when(i + NBUF < SB)
                def _():
                    pltpu.make_async_copy(rows.at[b],
                                          acc.at[dst_rows.at[i]],
                                          ssem.at[b]).wait()
                    pltpu.async_copy(t_hbm.at[src_rows.at[i + NBUF]],
                                     rows.at[b], gsem.at[b])

            return carry

        lax.fori_loop(0, SB // NBUF, body, 0)
        for b in range(NBUF):
            pltpu.make_async_copy(rows.at[b], acc.at[dst_rows.at[0]],
                                  ssem.at[b]).wait()

    plsc.subcore_barrier()
    pltpu.sync_copy(acc.at[pl.ds(rbase, RPT)],
                    out_hbm.at[c].at[pl.ds(rbase, RPT)])


R = 10000
GRID = N // R

_PREC = lax.Precision.DEFAULT


def _tc_scale_body(degp_ref, x_ref, t1_ref, dinv_ref):
    deg = degp_ref[:, 0] + degp_ref[:, 1] + 1.0
    dinv = lax.rsqrt(deg)
    dinv_ref[...] = dinv[:, None]
    t1_ref[...] = dinv[:, None] * x_ref[...]


def _tc_scale(degp_t, x):
    return pl.pallas_call(
        _tc_scale_body,
        grid=(GRID,),
        in_specs=[
            pl.BlockSpec((R, 2), lambda i: (i, 0)),
            pl.BlockSpec((R, D), lambda i: (i, 0)),
        ],
        out_specs=[
            pl.BlockSpec((R, D), lambda i: (i, 0)),
            pl.BlockSpec((R, 1), lambda i: (i, 0)),
        ],
        out_shape=[
            jax.ShapeDtypeStruct((N, D), jnp.float32),
            jax.ShapeDtypeStruct((N, 1), jnp.float32),
        ],
    )(degp_t, x)


def _tc_mid_body(y1p_ref, t1_ref, dinv_ref, w1_ref, b1_ref, w2_ref, t2_ref):
    dinv = dinv_ref[...]
    z1 = dinv * (y1p_ref[0] + y1p_ref[1] + t1_ref[...])
    h = jnp.maximum(
        jnp.dot(z1, w1_ref[...], preferred_element_type=jnp.float32,
                precision=_PREC) + b1_ref[...],
        0.0)
    s2 = jnp.dot(h, w2_ref[...], preferred_element_type=jnp.float32,
                 precision=_PREC)
    t2_ref[...] = dinv * s2


def _tc_mid(y1p, t1, dinv, W1, b1, W2):
    return pl.pallas_call(
        _tc_mid_body,
        grid=(GRID,),
        in_specs=[
            pl.BlockSpec((2, R, D), lambda i: (0, i, 0)),
            pl.BlockSpec((R, D), lambda i: (i, 0)),
            pl.BlockSpec((R, 1), lambda i: (i, 0)),
            pl.BlockSpec((D, 2 * D), lambda i: (0, 0)),
            pl.BlockSpec((1, 2 * D), lambda i: (0, 0)),
            pl.BlockSpec((2 * D, D), lambda i: (0, 0)),
        ],
        out_specs=pl.BlockSpec((R, D), lambda i: (i, 0)),
        out_shape=jax.ShapeDtypeStruct((N, D), jnp.float32),
    )(y1p, t1, dinv, W1, b1, W2)


def _tc_out_body(y2p_ref, t2_ref, dinv_ref, b2_ref, out_ref):
    out_ref[...] = dinv_ref[...] * (y2p_ref[0] + y2p_ref[1] + t2_ref[...]) \
        + b2_ref[...]


def _tc_out(y2p, t2, dinv, b2):
    return pl.pallas_call(
        _tc_out_body,
        grid=(GRID,),
        in_specs=[
            pl.BlockSpec((2, R, D), lambda i: (0, i, 0)),
            pl.BlockSpec((R, D), lambda i: (i, 0)),
            pl.BlockSpec((R, 1), lambda i: (i, 0)),
            pl.BlockSpec((1, D), lambda i: (0, 0)),
        ],
        out_specs=pl.BlockSpec((R, D), lambda i: (i, 0)),
        out_shape=jax.ShapeDtypeStruct((N, D), jnp.float32),
    )(y2p, t2, dinv, b2)


def kernel(x, edge_index, W1, b1, W2, b2):
    src2 = edge_index[0].astype(jnp.int32).reshape(NCHUNK, CH)
    dst2 = edge_index[1].astype(jnp.int32).reshape(NCHUNK, CH)
    npadc = NPAD - NCHUNK
    pad_flat = jnp.arange(npadc * CH, dtype=jnp.int32)
    src2 = jnp.concatenate(
        [src2, ((pad_flat * 131) % N).reshape(npadc, CH)])
    dst2 = jnp.concatenate(
        [dst2, (N + pad_flat % (N2 - N)).reshape(npadc, CH)])
    zeros_nd = jnp.zeros((N2, D), jnp.float32)
    zeros_n = jnp.zeros((N2,), jnp.float32)
    ones_ch = jnp.ones((CH,), jnp.float32)

    degp = _sc_deg(dst2, ones_ch, zeros_n)
    t1, dinv = _tc_scale(degp.T, x)
    y1p = _sc_msg(t1, src2, dst2, zeros_nd)
    t2 = _tc_mid(y1p, t1, dinv, W1, b1.reshape(1, -1), W2)
    y2p = _sc_msg(t2, src2, dst2, zeros_nd)
    return _tc_out(y2p, t2, dinv, b2.reshape(1, -1))
